# baseline (speedup 1.0000x reference)
import jax
import jax.numpy as jnp
from jax.experimental import pallas as pl
from jax.experimental.pallas import tpu as pltpu

E = 8
TOP_K = 2
H = 1024
I = 768


def _moe_dense_kernel(x_ref, aff_ref, idx_ref, gu_ref, dw_ref, out_ref,
                      w_ref, xb_ref, hs_ref):
    e = pl.program_id(0)

    @pl.when(e == 0)
    def _():
        idx = idx_ref[...]
        aff = aff_ref[...]
        lane = jax.lax.broadcasted_iota(jnp.int32, (1, E), 1)
        m0 = (idx[:, 0:1] == lane).astype(jnp.float32)
        m1 = (idx[:, 1:2] == lane).astype(jnp.float32)
        a0 = jnp.sum(m0 * aff, axis=1, keepdims=True)
        a1 = jnp.sum(m1 * aff, axis=1, keepdims=True)
        inv = 1.0 / (a0 + a1 + 1e-9)
        w_ref[:, 0:1] = a0 * inv
        w_ref[:, 1:2] = a1 * inv
        xb_ref[...] = x_ref[...].astype(jnp.bfloat16)

    idx = idx_ref[...]
    w = ((idx[:, 0:1] == e) * w_ref[:, 0:1]
         + (idx[:, 1:2] == e) * w_ref[:, 1:2])

    x = xb_ref[...]
    IT = 256
    for kt in range(I // IT):
        wg = gu_ref[0][:, kt * IT:(kt + 1) * IT].astype(jnp.bfloat16)
        wu = gu_ref[0][:, I + kt * IT:I + (kt + 1) * IT].astype(jnp.bfloat16)
        g = jnp.dot(x, wg, preferred_element_type=jnp.float32)
        u = jnp.dot(x, wu, preferred_element_type=jnp.float32)
        hs_ref[:, kt * IT:(kt + 1) * IT] = (
            jax.nn.sigmoid(g) * g * u * w).astype(jnp.bfloat16)

    wd = dw_ref[0].astype(jnp.bfloat16)
    y = jnp.dot(hs_ref[...], wd, preferred_element_type=jnp.float32)

    @pl.when(e == 0)
    def _():
        out_ref[...] = y

    @pl.when(e != 0)
    def _():
        out_ref[...] += y


@jax.jit
def kernel(hidden_states, expert_affinities, expert_indices, seq_len,
           gate_up_proj, down_proj):
    del seq_len
    T = hidden_states.shape[0]

    out = pl.pallas_call(
        _moe_dense_kernel,
        grid=(E,),
        in_specs=[
            pl.BlockSpec((T, H), lambda e: (0, 0)),
            pl.BlockSpec((T, E), lambda e: (0, 0)),
            pl.BlockSpec((T, TOP_K), lambda e: (0, 0)),
            pl.BlockSpec((1, H, 2 * I), lambda e: (e, 0, 0)),
            pl.BlockSpec((1, I, H), lambda e: (e, 0, 0)),
        ],
        out_specs=pl.BlockSpec((T, H), lambda e: (0, 0)),
        out_shape=jax.ShapeDtypeStruct((T, H), jnp.float32),
        scratch_shapes=[
            pltpu.VMEM((T, TOP_K), jnp.float32),
            pltpu.VMEM((T, H), jnp.bfloat16),
            pltpu.VMEM((T, I), jnp.bfloat16),
        ],
        compiler_params=pltpu.CompilerParams(
            dimension_semantics=("arbitrary",),
        ),
    )(hidden_states, expert_affinities, expert_indices,
      gate_up_proj, down_proj)
    return out
